# baseline (speedup 1.0000x reference)
import dataclasses
import functools

import jax
import jax.numpy as jnp
from jax import lax
from jax.experimental import pallas as pl
from jax.experimental.pallas import tpu as pltpu
from jax.experimental.pallas import tpu_sc as plsc

_LAYERS = 64
_BRANCHES = 8
_LANES = 16
_TILES = _LAYERS // _LANES
_SLICE = _LANES * _BRANCHES


def _log_1_to_8(s):
    bits = plsc.bitcast(s, jnp.int32)
    e = lax.shift_right_logical(bits, 23) - 127
    mant = plsc.bitcast(
        lax.bitwise_or(lax.bitwise_and(bits, 0x7FFFFF), 0x3F800000),
        jnp.float32)
    big = mant > 1.4142135
    mant = jnp.where(big, mant * 0.5, mant)
    e = e + jnp.where(big, 1, 0)
    r = (mant - 1.0) / (mant + 1.0)
    r2 = r * r
    poly = 2.0 * r * (1.0 + r2 * (0.33333333 + r2 * (0.2 + r2 * (0.14285714
                      + r2 * 0.11111111))))
    return e.astype(jnp.float32) * 0.69314718 + poly


def _sc_body(a_hbm, g_hbm, arc_hbm, lp_hbm, ent_hbm,
             a_v, g_v, arc_v, lp_v, ent_v):
    wid = lax.axis_index("s") * 2 + lax.axis_index("c")

    @pl.when(wid < _TILES)
    def _():
        base = wid * _SLICE
        pltpu.sync_copy(a_hbm.at[pl.ds(base, _SLICE)], a_v)
        pltpu.sync_copy(g_hbm.at[pl.ds(base, _SLICE)], g_v)

        lane = lax.broadcasted_iota(jnp.int32, (_LANES,), 0) * _BRANCHES
        a = [plsc.load_gather(a_v, [lane + b]) for b in range(_BRANCHES)]
        g = [plsc.load_gather(g_v, [lane + b]) for b in range(_BRANCHES)]

        m = a[0]
        for b in range(1, _BRANCHES):
            m = jnp.maximum(m, a[b])
        x = [a[b] - m for b in range(_BRANCHES)]

        best_key = a[0] + g[0]
        best_idx = jnp.full((_LANES,), 0, jnp.int32)
        best_x = x[0]
        for b in range(1, _BRANCHES):
            key = a[b] + g[b]
            take = key > best_key
            best_key = jnp.where(take, key, best_key)
            best_idx = jnp.where(take, b, best_idx)
            best_x = jnp.where(take, x[b], best_x)

        e0 = [jnp.exp(xb) for xb in x]
        s = e0[0]
        t = e0[0] * x[0]
        for b in range(1, _BRANCHES):
            s = s + e0[b]
            t = t + e0[b] * x[b]
        ln_s = _log_1_to_8(s)

        arc_v[...] = best_idx
        lp_v[...] = best_x - ln_s
        ent_v[...] = ln_s - t / s

        out = wid * _LANES
        pltpu.sync_copy(arc_v, arc_hbm.at[pl.ds(out, _LANES)])
        pltpu.sync_copy(lp_v, lp_hbm.at[pl.ds(out, _LANES)])
        pltpu.sync_copy(ent_v, ent_hbm.at[pl.ds(out, _LANES)])


@jax.jit
def kernel(alpha, gumbel):
    a = alpha.reshape(_LAYERS * _BRANCHES)
    g = gumbel.reshape(_LAYERS * _BRANCHES)
    mesh = plsc.VectorSubcoreMesh(core_axis_name="c", subcore_axis_name="s")
    cp = pltpu.CompilerParams()
    if "needs_layout_passes" in pltpu.CompilerParams.__dataclass_fields__:
        cp = dataclasses.replace(cp, needs_layout_passes=False)
    fn = pl.kernel(
        _sc_body,
        out_type=(
            jax.ShapeDtypeStruct((_LAYERS,), jnp.int32),
            jax.ShapeDtypeStruct((_LAYERS,), jnp.float32),
            jax.ShapeDtypeStruct((_LAYERS,), jnp.float32),
        ),
        mesh=mesh,
        scratch_types=[
            pltpu.VMEM((_SLICE,), jnp.float32),
            pltpu.VMEM((_SLICE,), jnp.float32),
            pltpu.VMEM((_LANES,), jnp.int32),
            pltpu.VMEM((_LANES,), jnp.float32),
            pltpu.VMEM((_LANES,), jnp.float32),
        ],
        compiler_params=cp,
    )
    arcs, lp, ent = fn(a, g)
    return (arcs.reshape(1, _LAYERS),
            lp.reshape(1, _LAYERS),
            ent.reshape(1, _LAYERS))


# speedup vs baseline: 11.0955x; 11.0955x over previous
import jax
import jax.numpy as jnp
from jax import lax
from jax.experimental import pallas as pl

_LAYERS = 64
_BRANCHES = 8


def _tc_body(a_ref, g_ref, arc_ref, lp_ref, ent_ref):
    a = a_ref[...]
    g = g_ref[...]
    m = jnp.max(a, axis=0, keepdims=True)
    x = a - m
    e = jnp.exp(x)
    s = jnp.sum(e, axis=0, keepdims=True)
    ln_s = jnp.log(s)
    ent_ref[...] = ln_s - jnp.sum(e * x, axis=0, keepdims=True) / s

    key = a + g
    kmax = jnp.max(key, axis=0, keepdims=True)
    idx = lax.broadcasted_iota(jnp.int32, (_BRANCHES, _LAYERS), 0)
    arg = jnp.min(jnp.where(key == kmax, idx, _BRANCHES), axis=0, keepdims=True)
    arc_ref[...] = arg
    x_sel = jnp.sum(jnp.where(idx == arg, x, 0.0), axis=0, keepdims=True)
    lp_ref[...] = x_sel - ln_s


@jax.jit
def kernel(alpha, gumbel):
    a_t = jnp.transpose(alpha.reshape(_LAYERS, _BRANCHES))
    g_t = jnp.transpose(gumbel.reshape(_LAYERS, _BRANCHES))
    out = pl.pallas_call(
        _tc_body,
        out_shape=(
            jax.ShapeDtypeStruct((1, _LAYERS), jnp.int32),
            jax.ShapeDtypeStruct((1, _LAYERS), jnp.float32),
            jax.ShapeDtypeStruct((1, _LAYERS), jnp.float32),
        ),
    )(a_t, g_t)
    return out
